# Initial kernel scaffold; baseline (speedup 1.0000x reference)
import jax
import jax.numpy as jnp
from jax.experimental import pallas as pl


def kernel(x, edge_index, enc_params, lin_W, lin_b, dec_params):
    raise NotImplementedError("write your pallas kernel here")


# speedup vs baseline: 14.6666x; 14.6666x over previous
import functools

import jax
import jax.numpy as jnp
from jax import lax
from jax.experimental import pallas as pl
from jax.experimental.pallas import tpu as pltpu
from jax.experimental.pallas import tpu_sc as plsc

F32 = jnp.float32
I32 = jnp.int32

NSC = 2
NT = 16
LANES = 16
RB = 64
WB = 64
BR = 1024
NEG = -1e30


def _leaky(v):
    return jnp.where(v > 0, v, 0.2 * v)


def _dense_body(mode, x_ref, *refs):
    i = pl.program_id(0)
    if mode == "first":
        (w_ref, a2_ref, h_ref, as_ref, ad_ref, m_ref) = refs
        z = x_ref[...]
    elif mode == "elu":
        (b_ref, w_ref, a2_ref, h_ref, as_ref, ad_ref, m_ref) = refs
        p = x_ref[0] + x_ref[1] + b_ref[0:1, :]
        z = jnp.where(p > 0, p, jnp.exp(p) - 1.0)
    elif mode == "lin":
        (b_ref, lw_ref, lb_ref, w_ref, a2_ref, h_ref, as_ref,
         ad_ref, m_ref) = refs
        p = x_ref[0] + x_ref[1] + b_ref[0:1, :]
        z = jnp.dot(p, lw_ref[...], preferred_element_type=F32) + lb_ref[0:1, :]
    else:
        (b_ref, o_ref) = refs
        o_ref[...] = x_ref[0] + x_ref[1] + b_ref[0:1, :]
        return
    h = jnp.dot(z, w_ref[...], preferred_element_type=F32)
    h_ref[...] = h
    asv = jnp.sum(h * a2_ref[0:1, :], axis=1)
    adv = jnp.sum(h * a2_ref[1:2, :], axis=1)
    as_ref[0, :] = asv
    ad_ref[0, :] = adv

    @pl.when(i == 0)
    def _():
        m_ref[...] = jnp.full((1, LANES), NEG, F32)

    m_ref[...] = jnp.maximum(m_ref[...], jnp.max(asv))


def _dense_stage(mode, npd, d, x, *args, interpret=False):
    grid = (npd // BR,)
    full = lambda shape: pl.BlockSpec(shape, lambda i: tuple(0 for _ in shape))
    p_spec = pl.BlockSpec((NSC, BR, d), lambda i: (0, i, 0))
    if mode == "final":
        in_specs = [p_spec, full((1, d))]
        out_shape = jax.ShapeDtypeStruct((npd, d), F32)
        out_specs = pl.BlockSpec((BR, d), lambda i: (i, 0))
    else:
        if mode == "first":
            x_spec = pl.BlockSpec((BR, d), lambda i: (i, 0))
            w_specs = [full((d, d)), full((2, d))]
        elif mode == "elu":
            x_spec = p_spec
            w_specs = [full((1, d)), full((d, d)), full((2, d))]
        else:
            x_spec = p_spec
            w_specs = [full((1, d)), full((d, d)), full((1, d)),
                       full((d, d)), full((2, d))]
        in_specs = [x_spec] + w_specs
        out_shape = (
            jax.ShapeDtypeStruct((npd, d), F32),
            jax.ShapeDtypeStruct((1, npd), F32),
            jax.ShapeDtypeStruct((1, npd), F32),
            jax.ShapeDtypeStruct((1, LANES), F32),
        )
        out_specs = (
            pl.BlockSpec((BR, d), lambda i: (i, 0)),
            pl.BlockSpec((1, BR), lambda i: (0, i)),
            pl.BlockSpec((1, BR), lambda i: (0, i)),
            pl.BlockSpec((1, LANES), lambda i: (0, 0)),
        )
    return pl.pallas_call(
        functools.partial(_dense_body, mode),
        grid=grid,
        in_specs=in_specs,
        out_specs=out_specs,
        out_shape=out_shape,
        interpret=interpret,
    )(x, *args)


def _sc_agg_call(npd, d, e_pad, interpret=False):
    nblk = e_pad // (NSC * NT * RB)
    ec2 = nblk * RB
    half_n = npd // 2
    rows_den = npd // NT
    rows_out = half_n // NT
    nqo = rows_out // WB

    def body(h_h, as_h, ad_h, mg_h, src_h, dst_h, out_h,
             den_s, out_s, src_v, dst_v, asl, adl, denl,
             exb, alb, idxb0, idxb1, didxb0, didxb1, stag0, stag1,
             mgv, zb, gs0, gs1, ss0, ss1):
        c = lax.axis_index("c")
        t = lax.axis_index("s")

        def load_half(w):
            pltpu.sync_copy(src_h.at[pl.ds(w * ec2, ec2)],
                            src_v.at[pl.ds(0, ec2)])
            pltpu.sync_copy(dst_h.at[pl.ds(w * ec2, ec2)], dst_v)
            for k in range(2 * RB // LANES):
                src_v[pl.ds(ec2 + k * LANES, LANES)] = jnp.zeros((LANES,),
                                                                 I32)

        pltpu.sync_copy(as_h, asl)
        pltpu.sync_copy(ad_h, adl)
        pltpu.sync_copy(mg_h, mgv)
        asl[pl.ds(npd - LANES, LANES)] = jnp.full((LANES,), NEG, F32)

        def zloop(i, u):
            zb[pl.ds(i * LANES, LANES)] = jnp.zeros((LANES,), F32)
            return u
        lax.fori_loop(0, rows_den // LANES, zloop, 0)
        pltpu.sync_copy(zb, den_s.at[pl.ds(t * rows_den, rows_den)])

        def zstag(i, u):
            for j in range(d // LANES):
                stag0[i, pl.ds(j * LANES, LANES)] = jnp.zeros((LANES,), F32)
            return u
        lax.fori_loop(0, RB, zstag, 0)

        def zero_out(q, u):
            pltpu.sync_copy(stag0.at[pl.ds(0, WB)],
                            out_s.at[pl.ds(t * rows_out + q * WB, WB)])
            return u
        lax.fori_loop(0, nqo, zero_out, 0)

        mg = mgv[pl.ds(0, LANES)][0]

        def edge_ex(off):
            sv = src_v[pl.ds(off, LANES)]
            dv = dst_v[pl.ds(off, LANES)]
            av = plsc.load_gather(asl, [sv])
            bv = plsc.load_gather(adl, [dv])
            e = _leaky(av + bv)
            mb = _leaky(bv + mg)
            return dv, jnp.exp(e - mb)

        def p1_compute(b, exb, didxb):
            for k in range(RB // LANES):
                off = b * RB + k * LANES
                dv, ex = edge_ex(off)
                exb[pl.ds(k * LANES, LANES)] = ex
                didxb[pl.ds(k * LANES, LANES)] = dv

        for half in range(2):
            load_half(NT * half + t)
            if half == 0:
                plsc.subcore_barrier()

            p1_compute(0, exb, didxb0)
            pltpu.async_copy(exb, den_s.at[didxb0], ss0, add=True)
            p1_compute(1, alb, didxb1)
            pltpu.async_copy(alb, den_s.at[didxb1], ss1, add=True)

            def p1(b2, u):
                b0 = 2 * b2
                pltpu.make_async_copy(exb, den_s.at[didxb0], ss0).wait()
                p1_compute(b0, exb, didxb0)
                pltpu.async_copy(exb, den_s.at[didxb0], ss0, add=True)
                pltpu.make_async_copy(alb, den_s.at[didxb1], ss1).wait()
                p1_compute(b0 + 1, alb, didxb1)
                pltpu.async_copy(alb, den_s.at[didxb1], ss1, add=True)
                return u
            lax.fori_loop(1, nblk // 2, p1, 0)
            pltpu.make_async_copy(exb, den_s.at[didxb0], ss0).wait()
            pltpu.make_async_copy(alb, den_s.at[didxb1], ss1).wait()
        plsc.subcore_barrier()
        pltpu.sync_copy(den_s, denl)

        load_half(NT * c + t)
        bufs = ((idxb0, didxb0, stag0, gs0, ss0),
                (idxb1, didxb1, stag1, gs1, ss1))

        def build_idx(b, idxb):
            for k in range(RB // LANES):
                off = b * RB + k * LANES
                idxb[pl.ds(k * LANES, LANES)] = src_v[pl.ds(off, LANES)]

        for rhalf in (0, 1):
            lo = rhalf * half_n

            def alpha_scale(b, didxb, stag):
                def grp(k, u2):
                    off = b * RB + k * LANES
                    dv, ex = edge_ex(off)
                    den = plsc.load_gather(denl, [dv])
                    al = ex / (den + 1e-16)
                    rel = dv - lo
                    inr = (rel >= 0) & (rel < half_n)
                    al = jnp.where(inr, al, 0.0)
                    didxb[pl.ds(k * LANES, LANES)] = jnp.clip(
                        rel, 0, half_n - 1)
                    for uu in range(LANES):
                        a = al[uu]
                        ii = k * LANES + uu
                        for j in range(d // LANES):
                            stag[ii, pl.ds(j * LANES, LANES)] = (
                                stag[ii, pl.ds(j * LANES, LANES)] * a)
                    return u2
                lax.fori_loop(0, RB // LANES, grp, 0)

            build_idx(0, idxb0)
            pltpu.make_async_copy(h_h.at[idxb0], stag0, gs0).start()
            build_idx(1, idxb1)
            pltpu.make_async_copy(h_h.at[idxb1], stag1, gs1).start()

            def p2(b2, u):
                b0 = 2 * b2
                for off, (idxb, didxb, stag, gs, ss) in ((0, bufs[0]),
                                                         (1, bufs[1])):
                    pltpu.make_async_copy(h_h.at[idxb], stag, gs).wait()
                    alpha_scale(b0 + off, didxb, stag)
                    pltpu.async_copy(stag, out_s.at[didxb], ss, add=True)
                for off, (idxb, didxb, stag, gs, ss) in ((2, bufs[0]),
                                                         (3, bufs[1])):
                    pltpu.make_async_copy(stag, out_s.at[didxb], ss).wait()
                    build_idx(b0 + off, idxb)
                    pltpu.make_async_copy(h_h.at[idxb], stag, gs).start()
                return u
            lax.fori_loop(0, nblk // 2, p2, 0)
            pltpu.make_async_copy(h_h.at[idxb0], stag0, gs0).wait()
            pltpu.make_async_copy(h_h.at[idxb1], stag1, gs1).wait()
            plsc.subcore_barrier()

            def wb(q, u):
                r0 = t * rows_out + q * WB
                pltpu.sync_copy(out_s.at[pl.ds(r0, WB)],
                                stag0.at[pl.ds(0, WB)])
                pltpu.sync_copy(stag0.at[pl.ds(0, WB)],
                                out_h.at[c, pl.ds(lo + r0, WB)])
                return u
            lax.fori_loop(0, nqo, wb, 0)

            if rhalf == 0:
                plsc.subcore_barrier()
                lax.fori_loop(0, RB, zstag, 0)
                lax.fori_loop(0, nqo, zero_out, 0)
                plsc.subcore_barrier()

    mesh = plsc.VectorSubcoreMesh(core_axis_name="c", subcore_axis_name="s",
                                  num_cores=NSC, num_subcores=NT)

    def wrapped(h, asv, adv, mg, src, dst):
        return pl.kernel(
            body,
            out_type=jax.ShapeDtypeStruct((NSC, npd, d), F32),
            mesh=mesh,
            scratch_types=[
                pltpu.VMEM_SHARED((npd,), F32),
                pltpu.VMEM_SHARED((half_n, d), F32),
                pltpu.VMEM((ec2 + 2 * RB,), I32),
                pltpu.VMEM((ec2,), I32),
                pltpu.VMEM((npd,), F32),
                pltpu.VMEM((npd,), F32),
                pltpu.VMEM((npd,), F32),
                pltpu.VMEM((RB,), F32),
                pltpu.VMEM((RB,), F32),
                pltpu.VMEM((RB,), I32),
                pltpu.VMEM((RB,), I32),
                pltpu.VMEM((RB,), I32),
                pltpu.VMEM((RB,), I32),
                pltpu.VMEM((RB, d), F32),
                pltpu.VMEM((RB, d), F32),
                pltpu.VMEM((LANES,), F32),
                pltpu.VMEM((rows_den,), F32),
                pltpu.SemaphoreType.DMA,
                pltpu.SemaphoreType.DMA,
                pltpu.SemaphoreType.DMA,
                pltpu.SemaphoreType.DMA,
            ],
            compiler_params=pltpu.CompilerParams(needs_layout_passes=False),
            interpret=interpret,
        )(h, asv, adv, mg, src, dst)

    return wrapped


def kernel(x, edge_index, enc_params, lin_W, lin_b, dec_params):
    n, d = x.shape
    e = edge_index.shape[1]
    npd = ((n + 2 * BR - 1) // (2 * BR)) * (2 * BR)
    blk_all = NSC * NT * RB
    nblk = (e + blk_all - 1) // blk_all
    nblk += nblk % 2
    e_pad = nblk * blk_all

    xp = jnp.zeros((npd, d), F32).at[:n, :].set(x)
    src = jnp.full((e_pad,), npd - 1, I32).at[:e].set(edge_index[0])
    dst = jnp.zeros((e_pad,), I32).at[:e].set(edge_index[1])

    sc = _sc_agg_call(npd, d, e_pad)

    def a2(p):
        return jnp.stack([p[1], p[2]])

    def run_sc(h, asv, adv, m):
        return sc(h, asv.reshape(npd), adv.reshape(npd), m.reshape(LANES),
                  src, dst)

    (w1, _, _, b1), (w2, _, _, b2) = enc_params
    (w3, _, _, b3), (w4, _, _, b4) = dec_params

    h, asv, adv, m = _dense_stage("first", npd, d, xp, w1, a2(enc_params[0]))
    p1 = run_sc(h, asv, adv, m)
    h, asv, adv, m = _dense_stage("elu", npd, d, p1, b1.reshape(1, d),
                                  w2, a2(enc_params[1]))
    p2 = run_sc(h, asv, adv, m)
    h, asv, adv, m = _dense_stage("lin", npd, d, p2, b2.reshape(1, d),
                                  lin_W, lin_b.reshape(1, d), w3,
                                  a2(dec_params[0]))
    p3 = run_sc(h, asv, adv, m)
    h, asv, adv, m = _dense_stage("elu", npd, d, p3, b3.reshape(1, d),
                                  w4, a2(dec_params[1]))
    p4 = run_sc(h, asv, adv, m)
    out = _dense_stage("final", npd, d, p4, b4.reshape(1, d))
    return out[:n]
